# baseline (speedup 1.0000x reference)
import jax
import jax.numpy as jnp
from jax.experimental import pallas as pl
from jax.experimental.pallas import tpu as pltpu

T = 2048
D = 2048
E = 8
R = 32
SCALING = 64 / 32
ER = E * R

TILE_T = 512
TILE_K = 512
NSLAB = D // TILE_K


def _slab_copy(wb_hbm, wb_vmem, sem, k):
    return pltpu.make_async_copy(
        wb_hbm.at[pl.ds(k * TILE_K, TILE_K), :],
        wb_vmem.at[pl.ds(k * TILE_K, TILE_K), :],
        sem.at[k],
    )


def _fused_kernel(x_ref, wb_hbm, wg_ref, a2_hbm, b2_hbm, o_ref,
                  wb_vmem, a2_vmem, b2_vmem, sem, sem2):
    i = pl.program_id(0)

    @pl.when(i == 0)
    def _start_stream():
        pltpu.make_async_copy(a2_hbm, a2_vmem, sem2.at[0]).start()
        pltpu.make_async_copy(b2_hbm, b2_vmem, sem2.at[1]).start()
        for k in range(NSLAB):
            _slab_copy(wb_hbm, wb_vmem, sem, k).start()

    x = x_ref[...]
    logits = jnp.dot(x, wg_ref[...], preferred_element_type=jnp.float32)
    cols = jax.lax.broadcasted_iota(jnp.int32, logits.shape, 1)
    m1 = jnp.max(logits, axis=1, keepdims=True)
    i1 = jnp.min(jnp.where(logits == m1, cols, E), axis=1, keepdims=True)
    masked = jnp.where(cols == i1, -jnp.inf, logits)
    m2 = jnp.max(masked, axis=1, keepdims=True)
    i2 = jnp.min(jnp.where(masked == m2, cols, E), axis=1, keepdims=True)
    e2 = jnp.exp(m2 - m1)
    denom = 1.0 + e2
    w1 = 1.0 / denom
    w2 = e2 / denom

    @pl.when(i == 0)
    def _wait_factors():
        pltpu.make_async_copy(a2_hbm, a2_vmem, sem2.at[0]).wait()
        pltpu.make_async_copy(b2_hbm, b2_vmem, sem2.at[1]).wait()

    a = jnp.dot(x, a2_vmem[...], preferred_element_type=jnp.float32)
    ecol = jax.lax.broadcasted_iota(jnp.int32, a.shape, 1) // R
    w_cols = jnp.where(ecol == i1, w1, 0.0) + jnp.where(ecol == i2, w2, 0.0)
    moe = jnp.dot(a * w_cols, b2_vmem[...], preferred_element_type=jnp.float32)

    @pl.when(i == 0)
    def _base_streamed():
        o_ref[...] = moe * SCALING
        for k in range(NSLAB):
            _slab_copy(wb_hbm, wb_vmem, sem, k).wait()
            o_ref[...] += jnp.dot(
                x[:, k * TILE_K:(k + 1) * TILE_K],
                wb_vmem[pl.ds(k * TILE_K, TILE_K), :],
                preferred_element_type=jnp.float32)

    @pl.when(i > 0)
    def _base_resident():
        base = jnp.dot(x, wb_vmem[...], preferred_element_type=jnp.float32)
        o_ref[...] = base + moe * SCALING


@jax.jit
def kernel(hidden_states, W_base, W_gate, lora_A, lora_B):
    A_all = lora_A.reshape(ER, D).T
    B_all = lora_B.transpose(0, 2, 1).reshape(ER, D)

    grid = (T // TILE_T,)
    return pl.pallas_call(
        _fused_kernel,
        grid=grid,
        in_specs=[
            pl.BlockSpec((TILE_T, D), lambda i: (i, 0)),
            pl.BlockSpec(memory_space=pltpu.HBM),
            pl.BlockSpec((D, E), lambda i: (0, 0)),
            pl.BlockSpec(memory_space=pltpu.HBM),
            pl.BlockSpec(memory_space=pltpu.HBM),
        ],
        out_specs=pl.BlockSpec((TILE_T, D), lambda i: (i, 0)),
        out_shape=jax.ShapeDtypeStruct((T, D), jnp.float32),
        scratch_shapes=[
            pltpu.VMEM((D, D), jnp.float32),
            pltpu.VMEM((D, ER), jnp.float32),
            pltpu.VMEM((ER, D), jnp.float32),
            pltpu.SemaphoreType.DMA((NSLAB,)),
            pltpu.SemaphoreType.DMA((2,)),
        ],
    )(hidden_states, W_base, W_gate, A_all, B_all)


# speedup vs baseline: 1.1702x; 1.1702x over previous
import jax
import jax.numpy as jnp
from jax.experimental import pallas as pl
from jax.experimental.pallas import tpu as pltpu

T = 2048
D = 2048
E = 8
R = 32
SCALING = 64 / 32
ER = E * R

TILE_T = 512
TILE_K = 512
NSLAB = D // TILE_K


def _slab_copy(wb_hbm, wb_vmem, sem, k):
    return pltpu.make_async_copy(
        wb_hbm.at[pl.ds(k * TILE_K, TILE_K), :],
        wb_vmem.at[pl.ds(k * TILE_K, TILE_K), :],
        sem.at[k],
    )


def _fused_kernel(x_ref, wb_hbm, wg_ref, a2_ref, b2_ref, o_ref, wb_vmem, sem):
    i = pl.program_id(0)

    @pl.when(i == 0)
    def _start_stream():
        for k in range(NSLAB):
            _slab_copy(wb_hbm, wb_vmem, sem, k).start()

    x = x_ref[...]
    logits = jnp.dot(x, wg_ref[...], preferred_element_type=jnp.float32)
    cols = jax.lax.broadcasted_iota(jnp.int32, logits.shape, 1)
    m1 = jnp.max(logits, axis=1, keepdims=True)
    i1 = jnp.min(jnp.where(logits == m1, cols, E), axis=1, keepdims=True)
    masked = jnp.where(cols == i1, -jnp.inf, logits)
    m2 = jnp.max(masked, axis=1, keepdims=True)
    i2 = jnp.min(jnp.where(masked == m2, cols, E), axis=1, keepdims=True)
    e2 = jnp.exp(m2 - m1)
    denom = 1.0 + e2
    w1 = 1.0 / denom
    w2 = e2 / denom

    a = jnp.dot(x, a2_ref[...], preferred_element_type=jnp.float32)
    ecol = jax.lax.broadcasted_iota(jnp.int32, a.shape, 1) // R
    w_cols = jnp.where(ecol == i1, w1, 0.0) + jnp.where(ecol == i2, w2, 0.0)
    moe = jnp.dot(a * w_cols, b2_ref[...], preferred_element_type=jnp.float32)

    @pl.when(i == 0)
    def _base_streamed():
        o_ref[...] = moe * SCALING
        for k in range(NSLAB):
            _slab_copy(wb_hbm, wb_vmem, sem, k).wait()
            o_ref[...] += jnp.dot(
                x[:, k * TILE_K:(k + 1) * TILE_K],
                wb_vmem[pl.ds(k * TILE_K, TILE_K), :],
                preferred_element_type=jnp.float32)

    @pl.when(i > 0)
    def _base_resident():
        base = jnp.dot(x, wb_vmem[...], preferred_element_type=jnp.float32)
        o_ref[...] = base + moe * SCALING


@jax.jit
def kernel(hidden_states, W_base, W_gate, lora_A, lora_B):
    A_all = lora_A.reshape(ER, D).T
    B_all = lora_B.transpose(0, 2, 1).reshape(ER, D)

    grid = (T // TILE_T,)
    return pl.pallas_call(
        _fused_kernel,
        grid=grid,
        in_specs=[
            pl.BlockSpec((TILE_T, D), lambda i: (i, 0)),
            pl.BlockSpec(memory_space=pltpu.HBM),
            pl.BlockSpec((D, E), lambda i: (0, 0)),
            pl.BlockSpec((D, ER), lambda i: (0, 0)),
            pl.BlockSpec((ER, D), lambda i: (0, 0)),
        ],
        out_specs=pl.BlockSpec((TILE_T, D), lambda i: (i, 0)),
        out_shape=jax.ShapeDtypeStruct((T, D), jnp.float32),
        scratch_shapes=[
            pltpu.VMEM((D, D), jnp.float32),
            pltpu.SemaphoreType.DMA((NSLAB,)),
        ],
    )(hidden_states, W_base, W_gate, A_all, B_all)
